# baseline (speedup 1.0000x reference)
import functools

import jax
import jax.numpy as jnp
import numpy as np
from jax.experimental import pallas as pl
from jax.experimental.pallas import tpu as pltpu

B = 2
S = 2048
T = B * S
HIDDEN = 1024
FF = 4096
FF_S = FF // 2
E = 8
K = 2
AUX_COEF = 0.01
Z_COEF = 0.001

_INTERPRET = False


_RT_BLK = 1024


def _router_body(x_ref, wr_ref, combine_ref, aux_ref, z_ref, ent_ref, acc_ref,
                 sacc_ref):
    i = pl.program_id(0)
    nb = pl.num_programs(0)

    @pl.when(i == 0)
    def _():
        acc_ref[...] = jnp.zeros_like(acc_ref)
        sacc_ref[0] = 0.0
        sacc_ref[1] = 0.0

    x = x_ref[...]
    logits = jnp.dot(x, wr_ref[...], preferred_element_type=jnp.float32)
    m = jnp.max(logits, axis=-1, keepdims=True)
    ex = jnp.exp(logits - m)
    s = jnp.sum(ex, axis=-1, keepdims=True)
    probs = ex / s

    lanes = jax.lax.broadcasted_iota(jnp.int32, probs.shape, 1)
    i1 = jnp.argmax(probs, axis=-1)[:, None]
    v1 = jnp.max(probs, axis=-1, keepdims=True)
    masked = jnp.where(lanes == i1, -jnp.inf, probs)
    i2 = jnp.argmax(masked, axis=-1)[:, None]
    v2 = jnp.max(masked, axis=-1, keepdims=True)
    tot = v1 + v2
    oh1 = (lanes == i1).astype(jnp.float32)
    oh2 = (lanes == i2).astype(jnp.float32)
    combine_ref[...] = (v1 / tot) * oh1 + (v2 / tot) * oh2

    z = jnp.log(s[:, 0]) + m[:, 0]
    ent = -jnp.sum(probs * jnp.log(probs + 1e-9), axis=-1)
    acc_ref[0, :] += jnp.sum(oh1 + oh2, axis=0)
    acc_ref[1, :] += jnp.sum(probs, axis=0)
    sacc_ref[0] += jnp.sum(z * z)
    sacc_ref[1] += jnp.sum(ent)

    @pl.when(i == nb - 1)
    def _():
        frac = acc_ref[0, :] / T
        mean_prob = acc_ref[1, :] / T
        aux_ref[0, 0] = AUX_COEF * E * jnp.sum(frac * mean_prob)
        z_ref[0, 0] = Z_COEF * sacc_ref[0] / T
        ent_ref[0, 0] = sacc_ref[1] / T


def _run_router(flat, w_router):
    nb = T // _RT_BLK
    return pl.pallas_call(
        _router_body,
        grid=(nb,),
        in_specs=[
            pl.BlockSpec((_RT_BLK, HIDDEN), lambda i: (i, 0)),
            pl.BlockSpec((HIDDEN, E), lambda i: (0, 0)),
        ],
        out_specs=[
            pl.BlockSpec((_RT_BLK, E), lambda i: (i, 0)),
            pl.BlockSpec(memory_space=pltpu.SMEM),
            pl.BlockSpec(memory_space=pltpu.SMEM),
            pl.BlockSpec(memory_space=pltpu.SMEM),
        ],
        out_shape=[
            jax.ShapeDtypeStruct((T, E), jnp.float32),
            jax.ShapeDtypeStruct((1, 1), jnp.float32),
            jax.ShapeDtypeStruct((1, 1), jnp.float32),
            jax.ShapeDtypeStruct((1, 1), jnp.float32),
        ],
        scratch_shapes=[pltpu.VMEM((2, E), jnp.float32),
                        pltpu.SMEM((2,), jnp.float32)],
        interpret=_INTERPRET,
    )(flat, w_router)


_TB = 256
_FB = 512


def _dense_moe_body(x_ref, cmb_ref, wg_ref, wu_ref, wd_ref, out_ref, acc_ref):
    e = pl.program_id(1)
    f = pl.program_id(2)

    @pl.when((e == 0) & (f == 0))
    def _():
        acc_ref[...] = jnp.zeros_like(acc_ref)

    x = x_ref[...]
    g = jnp.dot(x, wg_ref[0], preferred_element_type=jnp.float32)
    u = jnp.dot(x, wu_ref[0], preferred_element_type=jnp.float32)
    h = (g * jax.nn.sigmoid(g)) * u
    lanes = jax.lax.broadcasted_iota(jnp.int32, (1, E), 1)
    w = jnp.sum(cmb_ref[...] * (lanes == e).astype(jnp.float32), axis=1,
                keepdims=True)
    acc_ref[...] += jnp.dot(h, wd_ref[0], preferred_element_type=jnp.float32) * w

    @pl.when((e == E - 1) & (f == pl.num_programs(2) - 1))
    def _():
        out_ref[...] = acc_ref[...]


def _run_dense_moe(flat, combine, wg, wu, wd):
    grid = (T // _TB, E, FF // _FB)
    return pl.pallas_call(
        _dense_moe_body,
        grid=grid,
        in_specs=[
            pl.BlockSpec((_TB, HIDDEN), lambda t, e, f: (t, 0)),
            pl.BlockSpec((_TB, E), lambda t, e, f: (t, 0)),
            pl.BlockSpec((1, HIDDEN, _FB), lambda t, e, f: (e, 0, f)),
            pl.BlockSpec((1, HIDDEN, _FB), lambda t, e, f: (e, 0, f)),
            pl.BlockSpec((1, _FB, HIDDEN), lambda t, e, f: (e, f, 0)),
        ],
        out_specs=pl.BlockSpec((_TB, HIDDEN), lambda t, e, f: (t, 0)),
        out_shape=jax.ShapeDtypeStruct((T, HIDDEN), jnp.float32),
        scratch_shapes=[pltpu.VMEM((_TB, HIDDEN), jnp.float32)],
        compiler_params=pltpu.CompilerParams(
            dimension_semantics=("parallel", "arbitrary", "arbitrary"),
        ),
        interpret=_INTERPRET,
    )(flat, combine, wg, wu, wd)


def _shared_body(x_ref, exp_ref, wg_ref, wu_ref, wd_ref, out_ref, acc_ref):
    f = pl.program_id(1)

    @pl.when(f == 0)
    def _():
        acc_ref[...] = jnp.zeros_like(acc_ref)

    x = x_ref[...]
    g = jnp.dot(x, wg_ref[...], preferred_element_type=jnp.float32)
    u = jnp.dot(x, wu_ref[...], preferred_element_type=jnp.float32)
    h = (g * jax.nn.sigmoid(g)) * u
    acc_ref[...] += jnp.dot(h, wd_ref[...], preferred_element_type=jnp.float32)

    @pl.when(f == pl.num_programs(1) - 1)
    def _():
        out_ref[...] = exp_ref[...] + 0.5 * acc_ref[...]


def _run_shared(flat, expert_out, wg_s, wu_s, wd_s):
    grid = (T // _TB, FF_S // _FB)
    return pl.pallas_call(
        _shared_body,
        grid=grid,
        in_specs=[
            pl.BlockSpec((_TB, HIDDEN), lambda t, f: (t, 0)),
            pl.BlockSpec((_TB, HIDDEN), lambda t, f: (t, 0)),
            pl.BlockSpec((HIDDEN, _FB), lambda t, f: (0, f)),
            pl.BlockSpec((HIDDEN, _FB), lambda t, f: (0, f)),
            pl.BlockSpec((_FB, HIDDEN), lambda t, f: (f, 0)),
        ],
        out_specs=pl.BlockSpec((_TB, HIDDEN), lambda t, f: (t, 0)),
        out_shape=jax.ShapeDtypeStruct((T, HIDDEN), jnp.float32),
        scratch_shapes=[pltpu.VMEM((_TB, HIDDEN), jnp.float32)],
        compiler_params=pltpu.CompilerParams(
            dimension_semantics=("parallel", "arbitrary"),
        ),
        interpret=_INTERPRET,
    )(flat, expert_out, wg_s, wu_s, wd_s)


def kernel(hidden_states, W_router, Wg, Wu, Wd, Wg_s, Wu_s, Wd_s):
    Bv, Sv, D = hidden_states.shape
    flat = hidden_states.reshape(-1, D)
    combine, aux, z, ent = _run_router(flat, W_router)
    expert_out = _run_dense_moe(flat, combine, Wg, Wu, Wd)
    out = _run_shared(flat, expert_out, Wg_s, Wu_s, Wd_s)
    return (out.reshape(Bv, Sv, D), aux[0, 0], z[0, 0], ent[0, 0])
